# Initial kernel scaffold; baseline (speedup 1.0000x reference)
import jax
import jax.numpy as jnp
from jax.experimental import pallas as pl


def kernel(x, action_emb):
    raise NotImplementedError("write your pallas kernel here")


# speedup vs baseline: 15.5130x; 15.5130x over previous
import functools

import jax
import jax.numpy as jnp
from jax import lax
from jax.experimental import pallas as pl
from jax.experimental.pallas import tpu as pltpu
from jax.experimental.pallas import tpu_sc as plsc

BATCH = 16384
SEQ = 20
EMBED_DIM = 32
HALF = 16

NUM_CORES = 2
NUM_SUBCORES = 16
NUM_WORKERS = NUM_CORES * NUM_SUBCORES
B_PER_W = BATCH // NUM_WORKERS
IDX_PER_W = B_PER_W * SEQ

CHUNK_B = 64
CHUNK_IDX = CHUNK_B * SEQ
NUM_CHUNKS = B_PER_W // CHUNK_B
GATHER_W = 128
GATHERS_PER_CHUNK = CHUNK_IDX // GATHER_W


def _sc_embed_sum(table, x_flat):
    mesh = plsc.VectorSubcoreMesh(core_axis_name="c", subcore_axis_name="s")

    @functools.partial(
        pl.kernel,
        out_type=jax.ShapeDtypeStruct((BATCH, EMBED_DIM), jnp.float32),
        mesh=mesh,
        compiler_params=pltpu.CompilerParams(use_tc_tiling_on_sc=False),
        scratch_types=[
            pltpu.VMEM((IDX_PER_W,), jnp.int32),
            pltpu.VMEM((CHUNK_IDX, EMBED_DIM), jnp.float32),
            pltpu.VMEM((CHUNK_IDX, EMBED_DIM), jnp.float32),
            pltpu.VMEM((B_PER_W, EMBED_DIM), jnp.float32),
            pltpu.SemaphoreType.DMA,
            pltpu.SemaphoreType.DMA,
        ],
    )
    def k(table_hbm, idx_hbm, out_hbm, idx_v, rows0, rows1, out_v, sem0, sem1):
        wid = lax.axis_index("s") * NUM_CORES + lax.axis_index("c")
        base_b = wid * B_PER_W
        base_i = wid * IDX_PER_W
        pltpu.sync_copy(idx_hbm.at[pl.ds(base_i, IDX_PER_W)], idx_v)

        rows = (rows0, rows1)
        sems = (sem0, sem1)

        def fire(c):
            buf, sem = rows[c % 2], sems[c % 2]
            cps = []
            for g in range(GATHERS_PER_CHUNK):
                off = c * CHUNK_IDX + g * GATHER_W
                cps.append(pltpu.async_copy(
                    table_hbm.at[idx_v.at[pl.ds(off, GATHER_W)]],
                    buf.at[pl.ds(g * GATHER_W, GATHER_W)],
                    sem))
            return cps

        pending = fire(0)
        for c in range(NUM_CHUNKS):
            for cp in pending:
                cp.wait()
            if c + 1 < NUM_CHUNKS:
                pending = fire(c + 1)
            buf = rows[c % 2]

            @pl.loop(0, CHUNK_B)
            def _(b, _c=c, _buf=buf):
                r0 = b * SEQ
                acc0 = _buf[r0, pl.ds(0, HALF)]
                acc1 = _buf[r0, pl.ds(HALF, HALF)]
                for l in range(1, SEQ):
                    acc0 = acc0 + _buf[r0 + l, pl.ds(0, HALF)]
                    acc1 = acc1 + _buf[r0 + l, pl.ds(HALF, HALF)]
                ob = _c * CHUNK_B + b
                out_v[ob, pl.ds(0, HALF)] = acc0
                out_v[ob, pl.ds(HALF, HALF)] = acc1

        pltpu.sync_copy(out_v, out_hbm.at[pl.ds(base_b, B_PER_W)])

    return k(table, x_flat)


def kernel(x, action_emb):
    x_flat = x.reshape(-1).astype(jnp.int32)
    return _sc_embed_sum(action_emb, x_flat)
